# Initial kernel scaffold; baseline (speedup 1.0000x reference)
import jax
import jax.numpy as jnp
from jax.experimental import pallas as pl


def kernel(x, pe, dim_idx):
    raise NotImplementedError("write your pallas kernel here")


# speedup vs baseline: 2.4804x; 2.4804x over previous
import jax
import jax.numpy as jnp
from jax import lax
from jax.experimental import pallas as pl
from jax.experimental.pallas import tpu as pltpu
from jax.experimental.pallas import tpu_sc as plsc

import functools


def _make_sc_gather(B, V, D, max_idx):
    info = plsc.get_sparse_core_info()
    NC, NS, L = info.num_cores, info.num_subcores, info.num_lanes
    NW = NC * NS
    assert B % NW == 0 and D % L == 0
    b_per_w = B // NW
    CHUNK = 128
    n_chunks = b_per_w // CHUNK

    mesh = plsc.VectorSubcoreMesh(core_axis_name="c", subcore_axis_name="s")

    @functools.partial(
        pl.kernel,
        mesh=mesh,
        out_type=jax.ShapeDtypeStruct((B, D), jnp.float32),
        scratch_types=[
            pltpu.VMEM((b_per_w,), jnp.float32),
            pltpu.VMEM((b_per_w,), jnp.int32),
            pltpu.VMEM((b_per_w, D), jnp.float32),
            pltpu.SemaphoreType.DMA,
        ],
    )
    def gather_kernel(pe_hbm, vals_hbm, out_hbm, vals_v, idx_v, rows_v, sem):
        wid = lax.axis_index("s") * NC + lax.axis_index("c")
        base = wid * b_per_w
        pltpu.sync_copy(vals_hbm.at[pl.ds(base, b_per_w)], vals_v)
        for i in range(b_per_w // L):
            v = vals_v[pl.ds(i * L, L)]
            idx = (v * 1000.0).astype(jnp.int32)
            idx_v[pl.ds(i * L, L)] = jnp.minimum(
                jnp.maximum(idx, 0), max_idx)
        copies = []
        for j in range(n_chunks):
            copies.append(pltpu.async_copy(
                pe_hbm.at[idx_v.at[pl.ds(j * CHUNK, CHUNK)]],
                rows_v.at[pl.ds(j * CHUNK, CHUNK)],
                sem,
            ))
        for c in copies:
            c.wait()
        pltpu.sync_copy(rows_v, out_hbm.at[pl.ds(base, b_per_w)])

    return gather_kernel


def kernel(x, pe, dim_idx):
    vals = x[:, dim_idx]
    B = x.shape[0]
    V, D = pe.shape
    fn = _make_sc_gather(B, V, D, V - 1)
    return fn(pe, vals)
